# Initial kernel scaffold; baseline (speedup 1.0000x reference)
import jax
import jax.numpy as jnp
from jax.experimental import pallas as pl


def kernel(coords, spatial_table, xyt_table, yzt_table, xzt_table):
    raise NotImplementedError("write your pallas kernel here")


# speedup vs baseline: 1.1276x; 1.1276x over previous
import functools
import numpy as np
import jax
import jax.numpy as jnp
from jax import lax
from jax.experimental import pallas as pl
from jax.experimental.pallas import tpu as pltpu
from jax.experimental.pallas import tpu_sc as plsc

LOG2T = 19
TSIZE = 1 << LOG2T
PRIME1 = -1640531535
PRIME2 = 805459861
NPTS = 131072
NCORE, NSUB = 2, 16
NWORK = NCORE * NSUB
PTILE = NPTS // NWORK
BCH = 512
NCHUNK = PTILE // BCH
OUTD = 128
NG = BCH // 16


def _scale_table():
    def mk(base, mx, nl):
        base = np.asarray(base, np.float64)
        mx = np.asarray(mx, np.float64)
        b = np.exp((np.log(mx) - np.log(base)) / (nl - 1))
        res = np.stack([np.maximum(np.floor(base * b ** l), 1.0)
                        for l in range(nl)])
        return res
    r_sp = mk([16, 16, 16], [512, 512, 512], 16)
    r_pr = mk([8, 8, 8], [32, 32, 16], 16)
    res = np.stack([r_sp, r_pr, r_pr, r_pr])
    return (res - 1.0).astype(np.float32)


_SCALES = _scale_table()
_SCALES_B = np.repeat(_SCALES.reshape(192, 1), 16, axis=1)

_ENC_COLS = ((0, 1, 2), (0, 1, 3), (1, 2, 3), (0, 2, 3))

_mesh = plsc.VectorSubcoreMesh(core_axis_name="c", subcore_axis_name="s")


@functools.partial(
    pl.kernel,
    out_type=jax.ShapeDtypeStruct((NPTS * OUTD,), jnp.float32),
    mesh=_mesh,
    scratch_types=(
        [pltpu.VMEM((BCH * 4,), jnp.float32),
         pltpu.VMEM((192, 16), jnp.float32),
         pltpu.VMEM((BCH * OUTD,), jnp.float32)]
        + [pltpu.VMEM((BCH,), jnp.int32) for _ in range(16)]
        + [pltpu.VMEM((BCH,), jnp.float32) for _ in range(16)]
        + [pltpu.SemaphoreType.DMA]
    ),
    compiler_params=pltpu.CompilerParams(needs_layout_passes=False),
)
def _sc_encode(xin_hbm, scales_hbm, t0, t1, t2, t3, out_hbm,
               xin_v, scales_v, out_v, *rest):
    idxb = rest[0:16]
    featb = rest[16:32]
    sem = rest[32]
    tables = (t0, t1, t2, t3)

    wid = lax.axis_index("s") * NCORE + lax.axis_index("c")
    iota = lax.iota(jnp.int32, 16)
    pat128 = iota * 128
    zero16 = jnp.zeros((16,), jnp.int32)
    p1v = jnp.full((16,), PRIME1, jnp.int32)
    p2v = jnp.full((16,), PRIME2, jnp.int32)
    mask2v = jnp.full((16,), (TSIZE - 1) << 1, jnp.int32)
    iota4 = iota * 4

    pltpu.sync_copy(scales_hbm, scales_v)
    pt0 = wid * PTILE

    def chunk_body(ci, _):
        base_pt = pt0 + ci * BCH
        pltpu.sync_copy(xin_hbm.at[pl.ds(base_pt * 4, BCH * 4)], xin_v)

        for e in range(4):
            tbl = tables[e]
            cols = _ENC_COLS[e]

            def lvl_body(l, __, e=e, tbl=tbl, cols=cols):
                srow = (e * 16 + l) * 3
                sx = scales_v[srow]
                sy = scales_v[srow + 1]
                sz = scales_v[srow + 2]
                lvlbase2 = l * (TSIZE * 2) + zero16

                def gen(g, ___):
                    p4 = iota4 + g * 64
                    ux = plsc.load_gather(xin_v, [p4 + cols[0]])
                    uy = plsc.load_gather(xin_v, [p4 + cols[1]])
                    uz = plsc.load_gather(xin_v, [p4 + cols[2]])
                    p0x = (ux * sx).astype(jnp.int32)
                    p0y = (uy * sy).astype(jnp.int32)
                    p0z = (uz * sz).astype(jnp.int32)
                    hx = (p0x, p0x + 1)
                    hy0 = p0y * p1v
                    hy = (hy0, hy0 + p1v)
                    hz0 = p0z * p2v
                    hz = (hz0, hz0 + p2v)
                    for c in range(8):
                        h = hx[c & 1] ^ hy[(c >> 1) & 1] ^ hz[(c >> 2) & 1]
                        h2 = (lax.shift_left(h, 1) & mask2v) + lvlbase2
                        idxb[2 * c][pl.ds(g * 16, 16)] = h2
                        idxb[2 * c + 1][pl.ds(g * 16, 16)] = h2 + 1
                    return 0

                lax.fori_loop(0, NG, gen, 0)

                descs = [pltpu.async_copy(tbl.at[idxb[k]], featb[k], sem)
                         for k in range(16)]
                for d in descs:
                    d.wait()

                coloff = e * 32 + l * 2

                def accum(g, ___):
                    p4 = iota4 + g * 64
                    ux = plsc.load_gather(xin_v, [p4 + cols[0]])
                    uy = plsc.load_gather(xin_v, [p4 + cols[1]])
                    uz = plsc.load_gather(xin_v, [p4 + cols[2]])
                    px = ux * sx
                    py = uy * sy
                    pz = uz * sz
                    fx = px - px.astype(jnp.int32).astype(jnp.float32)
                    fy = py - py.astype(jnp.int32).astype(jnp.float32)
                    fz = pz - pz.astype(jnp.int32).astype(jnp.float32)
                    gx = 1.0 - fx
                    gy = 1.0 - fy
                    gz = 1.0 - fz
                    acc0 = jnp.zeros((16,), jnp.float32)
                    acc1 = jnp.zeros((16,), jnp.float32)
                    for c in range(8):
                        w = ((fx if c & 1 else gx)
                             * (fy if c & 2 else gy)
                             * (fz if c & 4 else gz))
                        f0 = featb[2 * c][pl.ds(g * 16, 16)]
                        f1 = featb[2 * c + 1][pl.ds(g * 16, 16)]
                        acc0 = acc0 + w * f0
                        acc1 = acc1 + w * f1
                    dst = pat128 + (g * 2048 + coloff)
                    plsc.store_scatter(out_v, [dst], acc0)
                    plsc.store_scatter(out_v, [dst + 1], acc1)
                    return 0

                lax.fori_loop(0, NG, accum, 0)
                return 0

            lax.fori_loop(0, 16, lvl_body, 0)

        pltpu.sync_copy(out_v, out_hbm.at[pl.ds(base_pt * OUTD, BCH * OUTD)])
        return 0

    lax.fori_loop(0, NCHUNK, chunk_body, 0)


def kernel(coords, spatial_table, xyt_table, yzt_table, xzt_table):
    lat = coords[:, 0]
    lon = coords[:, 1]
    elev = coords[:, 2]
    t = coords[:, 3]
    earth_radius = 6371000.0
    lat_rad = jnp.deg2rad(lat)
    lon_rad = jnp.deg2rad(lon)
    radius = earth_radius + elev
    x = radius * jnp.cos(lat_rad) * jnp.cos(lon_rad)
    y = radius * jnp.cos(lat_rad) * jnp.sin(lon_rad)
    z = radius * jnp.sin(lat_rad)
    sp = jnp.stack([x, y, z], axis=-1)
    sp_min = jnp.min(sp, axis=0)
    sp_max = jnp.max(sp, axis=0)
    sp_n = (sp - sp_min) / (sp_max - sp_min)
    t_min = jnp.min(t)
    t_max = jnp.max(t)
    t_n = (t - t_min) / (t_max - t_min)
    t_s = (t_n * 2.0 - 1.0) * 0.9

    u = jnp.clip((sp_n + 1.0) / 2.0, 0.0, 1.0)
    ut = jnp.clip((t_s + 1.0) / 2.0, 0.0, 1.0)
    xin = jnp.concatenate([u, ut[:, None]], axis=1)

    scales_b = jnp.asarray(_SCALES_B)
    out = _sc_encode(
        xin.reshape(NPTS * 4), scales_b,
        spatial_table.reshape(16 * TSIZE * 2),
        xyt_table.reshape(16 * TSIZE * 2),
        yzt_table.reshape(16 * TSIZE * 2),
        xzt_table.reshape(16 * TSIZE * 2),
    )
    return out.reshape(NPTS, OUTD)


# speedup vs baseline: 1.1373x; 1.0086x over previous
import functools
import numpy as np
import jax
import jax.numpy as jnp
from jax import lax
from jax.experimental import pallas as pl
from jax.experimental.pallas import tpu as pltpu
from jax.experimental.pallas import tpu_sc as plsc

LOG2T = 19
TSIZE = 1 << LOG2T
PRIME1 = -1640531535
PRIME2 = 805459861
NPTS = 131072
NCORE, NSUB = 2, 16
NWORK = NCORE * NSUB
PTILE = NPTS // NWORK
BCH = 512
NCHUNK = PTILE // BCH
OUTD = 128
NG = BCH // 16


def _scale_table():
    def mk(base, mx, nl):
        base = np.asarray(base, np.float64)
        mx = np.asarray(mx, np.float64)
        b = np.exp((np.log(mx) - np.log(base)) / (nl - 1))
        res = np.stack([np.maximum(np.floor(base * b ** l), 1.0)
                        for l in range(nl)])
        return res
    r_sp = mk([16, 16, 16], [512, 512, 512], 16)
    r_pr = mk([8, 8, 8], [32, 32, 16], 16)
    res = np.stack([r_sp, r_pr, r_pr, r_pr])
    return (res - 1.0).astype(np.float32)


_SCALES = _scale_table()
_SCALES_B = np.repeat(_SCALES.reshape(192, 1), 16, axis=1)

_ENC_COLS = ((0, 1, 2), (0, 1, 3), (1, 2, 3), (0, 2, 3))

_mesh = plsc.VectorSubcoreMesh(core_axis_name="c", subcore_axis_name="s")


@functools.partial(
    pl.kernel,
    out_type=jax.ShapeDtypeStruct((NPTS * OUTD,), jnp.float32),
    mesh=_mesh,
    scratch_types=(
        [pltpu.VMEM((BCH * 4,), jnp.float32),
         pltpu.VMEM((192, 16), jnp.float32),
         pltpu.VMEM((BCH * OUTD,), jnp.float32)]
        + [pltpu.VMEM((2 * BCH,), jnp.int32) for _ in range(16)]
        + [pltpu.VMEM((2 * BCH,), jnp.float32) for _ in range(16)]
        + [pltpu.SemaphoreType.DMA]
    ),
    compiler_params=pltpu.CompilerParams(needs_layout_passes=False),
)
def _sc_encode(xin_hbm, scales_hbm, t0, t1, t2, t3, out_hbm,
               xin_v, scales_v, out_v, *rest):
    idxb = (rest[0:8], rest[8:16])
    featb = (rest[16:24], rest[24:32])
    sem = rest[32]
    tables = (t0, t1, t2, t3)

    wid = lax.axis_index("s") * NCORE + lax.axis_index("c")
    iota = lax.iota(jnp.int32, 16)
    pat128 = iota * 128
    zero16 = jnp.zeros((16,), jnp.int32)
    one16 = jnp.full((16,), 1, jnp.int32)
    p1v = jnp.full((16,), PRIME1, jnp.int32)
    p2v = jnp.full((16,), PRIME2, jnp.int32)
    mask2v = jnp.full((16,), (TSIZE - 1) << 1, jnp.int32)
    iota4 = iota * 4

    pltpu.sync_copy(scales_hbm, scales_v)
    pt0 = wid * PTILE

    def chunk_body(ci, _):
        base_pt = pt0 + ci * BCH
        pltpu.sync_copy(xin_hbm.at[pl.ds(base_pt * 4, BCH * 4)], xin_v)

        for e in range(4):
            tbl = tables[e]
            cols = _ENC_COLS[e]

            def gen(l, s, cols=cols):
                srow = jnp.int32(e * 48) + l * 3
                sx = scales_v[srow]
                sy = scales_v[srow + 1]
                sz = scales_v[srow + 2]
                lvlbase2 = l * (TSIZE * 2) + zero16

                def body(g, ___):
                    p4 = iota4 + g * 64
                    ux = plsc.load_gather(xin_v, [p4 + cols[0]])
                    uy = plsc.load_gather(xin_v, [p4 + cols[1]])
                    uz = plsc.load_gather(xin_v, [p4 + cols[2]])
                    p0x = (ux * sx).astype(jnp.int32)
                    p0y = (uy * sy).astype(jnp.int32)
                    p0z = (uz * sz).astype(jnp.int32)
                    hx = (p0x, p0x + 1)
                    hy0 = p0y * p1v
                    hy = (hy0, hy0 + p1v)
                    hz0 = p0z * p2v
                    hz = (hz0, hz0 + p2v)
                    for c in range(8):
                        h = hx[c & 1] ^ hy[(c >> 1) & 1] ^ hz[(c >> 2) & 1]
                        h2 = (lax.shift_left(h, 1) & mask2v) + lvlbase2
                        idxb[s][c][pl.ds(g * 16, 16)] = h2
                        idxb[s][c][pl.ds(BCH + g * 16, 16)] = h2 + 1
                    return 0

                lax.fori_loop(0, NG, body, 0)

            def fire(s, tbl=tbl):
                return [pltpu.async_copy(tbl.at[idxb[s][c]], featb[s][c], sem)
                        for c in range(8)]

            def drain(descs):
                for d in descs:
                    d.wait()

            def accum(l, s, cols=cols, e=e):
                srow = jnp.int32(e * 48) + l * 3
                sx = scales_v[srow]
                sy = scales_v[srow + 1]
                sz = scales_v[srow + 2]
                coloff = l * 2 + jnp.int32(e * 32)

                def body(g, ___):
                    p4 = iota4 + g * 64
                    ux = plsc.load_gather(xin_v, [p4 + cols[0]])
                    uy = plsc.load_gather(xin_v, [p4 + cols[1]])
                    uz = plsc.load_gather(xin_v, [p4 + cols[2]])
                    px = ux * sx
                    py = uy * sy
                    pz = uz * sz
                    fx = px - px.astype(jnp.int32).astype(jnp.float32)
                    fy = py - py.astype(jnp.int32).astype(jnp.float32)
                    fz = pz - pz.astype(jnp.int32).astype(jnp.float32)
                    gx = 1.0 - fx
                    gy = 1.0 - fy
                    gz = 1.0 - fz
                    acc0 = jnp.zeros((16,), jnp.float32)
                    acc1 = jnp.zeros((16,), jnp.float32)
                    for c in range(8):
                        w = ((fx if c & 1 else gx)
                             * (fy if c & 2 else gy)
                             * (fz if c & 4 else gz))
                        f0 = featb[s][c][pl.ds(g * 16, 16)]
                        f1 = featb[s][c][pl.ds(BCH + g * 16, 16)]
                        acc0 = acc0 + w * f0
                        acc1 = acc1 + w * f1
                    dst = pat128 + (g * 2048 + coloff)
                    plsc.store_scatter(out_v, [dst], acc0)
                    plsc.store_scatter(out_v, [dst + 1], acc1)
                    return 0

                lax.fori_loop(0, NG, body, 0)

            gen(jnp.int32(0), 0)
            dA = fire(0)
            dB = None

            def pipe(ll, ___):
                l0 = ll * 2
                gen(l0 + 1, 1)
                dB = fire(1)
                drain(dA)
                accum(l0, 0)
                gen(l0 + 2, 0)
                dA2 = fire(0)
                drain(dB)
                accum(l0 + 1, 1)
                return ___

            lax.fori_loop(0, 7, pipe, 0)
            gen(jnp.int32(15), 1)
            dB = fire(1)
            drain(dA)
            accum(jnp.int32(14), 0)
            drain(dB)
            accum(jnp.int32(15), 1)

        pltpu.sync_copy(out_v, out_hbm.at[pl.ds(base_pt * OUTD, BCH * OUTD)])
        return 0

    lax.fori_loop(0, NCHUNK, chunk_body, 0)


def kernel(coords, spatial_table, xyt_table, yzt_table, xzt_table):
    lat = coords[:, 0]
    lon = coords[:, 1]
    elev = coords[:, 2]
    t = coords[:, 3]
    earth_radius = 6371000.0
    lat_rad = jnp.deg2rad(lat)
    lon_rad = jnp.deg2rad(lon)
    radius = earth_radius + elev
    x = radius * jnp.cos(lat_rad) * jnp.cos(lon_rad)
    y = radius * jnp.cos(lat_rad) * jnp.sin(lon_rad)
    z = radius * jnp.sin(lat_rad)
    sp = jnp.stack([x, y, z], axis=-1)
    sp_min = jnp.min(sp, axis=0)
    sp_max = jnp.max(sp, axis=0)
    sp_n = (sp - sp_min) / (sp_max - sp_min)
    t_min = jnp.min(t)
    t_max = jnp.max(t)
    t_n = (t - t_min) / (t_max - t_min)
    t_s = (t_n * 2.0 - 1.0) * 0.9

    u = jnp.clip((sp_n + 1.0) / 2.0, 0.0, 1.0)
    ut = jnp.clip((t_s + 1.0) / 2.0, 0.0, 1.0)
    xin = jnp.concatenate([u, ut[:, None]], axis=1)

    scales_b = jnp.asarray(_SCALES_B)
    out = _sc_encode(
        xin.reshape(NPTS * 4), scales_b,
        spatial_table.reshape(16 * TSIZE * 2),
        xyt_table.reshape(16 * TSIZE * 2),
        yzt_table.reshape(16 * TSIZE * 2),
        xzt_table.reshape(16 * TSIZE * 2),
    )
    return out.reshape(NPTS, OUTD)
